# baseline (speedup 1.0000x reference)
import functools

import jax
import jax.numpy as jnp
from jax import lax
from jax.experimental import pallas as pl
from jax.experimental.pallas import tpu as pltpu
from jax.experimental.pallas import tpu_sc as plsc

VOCAB = 1000000
EMB = 64
BATCH = 4096
SEQ = 200
NTOK = BATCH * SEQ

NC = 2
NS = 16
NW = NC * NS
ROWS_W = BATCH // NW
PER_W = NTOK // NW

GDMA = 128
CHUNK = 512
N_GATH = CHUNK // GDMA
N_CHUNKS = PER_W // CHUNK
LANES = 16


@functools.partial(
    pl.kernel,
    mesh=plsc.VectorSubcoreMesh(core_axis_name="c", subcore_axis_name="s"),
    compiler_params=pltpu.CompilerParams(
        use_tc_tiling_on_sc=False, needs_layout_passes=False),
    out_type=jax.ShapeDtypeStruct((NTOK, EMB), jnp.float32),
    scratch_types=[
        pltpu.VMEM((ROWS_W, SEQ), jnp.int32),
        pltpu.VMEM((2, CHUNK), jnp.int32),
        pltpu.VMEM((2, CHUNK, EMB), jnp.float32),
        pltpu.SemaphoreType.DMA,
        pltpu.SemaphoreType.DMA,
    ],
)
def _emb_lookup(x_hbm, table_hbm, out_hbm, x_v, idx_v, rows_v, sem0, sem1):
    wid = lax.axis_index("s") * NC + lax.axis_index("c")
    tok_base = wid * PER_W
    sems = (sem0, sem1)

    pltpu.sync_copy(x_hbm.at[pl.ds(wid * ROWS_W, ROWS_W)], x_v)

    def stage_and_fire(g, b):
        for k in range(CHUNK // LANES):
            u = g * CHUNK + k * LANES + lax.iota(jnp.int32, LANES)
            ids = plsc.load_gather(x_v, [lax.div(u, SEQ), lax.rem(u, SEQ)])
            idx_v[b, pl.ds(k * LANES, LANES)] = ids
        for j in range(N_GATH):
            pltpu.async_copy(table_hbm.at[idx_v.at[b].at[pl.ds(j * GDMA, GDMA)]],
                             rows_v.at[b].at[pl.ds(j * GDMA, GDMA)], sems[b])

    def drain_gathers(b):
        for j in range(N_GATH):
            pltpu.make_async_copy(
                table_hbm.at[idx_v.at[b].at[pl.ds(j * GDMA, GDMA)]],
                rows_v.at[b].at[pl.ds(j * GDMA, GDMA)], sems[b]).wait()

    def store_out(g, b):
        pltpu.sync_copy(rows_v.at[b],
                        out_hbm.at[pl.ds(tok_base + g * CHUNK, CHUNK)])

    stage_and_fire(0, 0)
    stage_and_fire(1, 1)

    def body(p, carry):
        for b in range(2):
            g = 2 * p + b
            drain_gathers(b)
            store_out(g, b)
            stage_and_fire(g + 2, b)
        return carry

    lax.fori_loop(0, N_CHUNKS // 2 - 1, body, 0)

    for b in range(2):
        g = N_CHUNKS - 2 + b
        drain_gathers(b)
        store_out(g, b)


def kernel(x, table):
    out = _emb_lookup(x.astype(jnp.int32), table)
    return out.reshape(BATCH, SEQ, EMB)


# speedup vs baseline: 1.3870x; 1.3870x over previous
import functools

import jax
import jax.numpy as jnp
from jax import lax
from jax.experimental import pallas as pl
from jax.experimental.pallas import tpu as pltpu
from jax.experimental.pallas import tpu_sc as plsc

VOCAB = 1000000
EMB = 64
PAD_EMB = 128
BATCH = 4096
SEQ = 200
NTOK = BATCH * SEQ

NC = 2
NS = 16
NW = NC * NS
ROWS_W = BATCH // NW
PER_W = NTOK // NW

GDMA = 128
CHUNK = 256
N_GATH = CHUNK // GDMA
N_CHUNKS = PER_W // CHUNK
LANES = 16
HALF = CHUNK // 2


@functools.partial(
    pl.kernel,
    mesh=plsc.VectorSubcoreMesh(core_axis_name="c", subcore_axis_name="s"),
    compiler_params=pltpu.CompilerParams(needs_layout_passes=False),
    out_type=jax.ShapeDtypeStruct((NTOK, EMB), jnp.float32),
    scratch_types=[
        pltpu.VMEM((ROWS_W, SEQ), jnp.int32),
        pltpu.VMEM((2, CHUNK), jnp.int32),
        pltpu.VMEM((2, CHUNK, PAD_EMB), jnp.float32),
        pltpu.VMEM((HALF, EMB), jnp.float32),
        pltpu.SemaphoreType.DMA,
        pltpu.SemaphoreType.DMA,
    ],
)
def _emb_lookup(x_hbm, table_hbm, out_hbm, x_v, idx_v, rows_v, rows64_v,
                sem0, sem1):
    wid = lax.axis_index("s") * NC + lax.axis_index("c")
    tok_base = wid * PER_W
    sems = (sem0, sem1)

    pltpu.sync_copy(x_hbm.at[pl.ds(wid * ROWS_W, ROWS_W)], x_v)

    def stage_and_fire(g, b):
        for k in range(CHUNK // LANES):
            u = g * CHUNK + k * LANES + lax.iota(jnp.int32, LANES)
            ids = plsc.load_gather(x_v, [lax.div(u, SEQ), lax.rem(u, SEQ)])
            idx_v[b, pl.ds(k * LANES, LANES)] = ids
        for j in range(N_GATH):
            pltpu.async_copy(table_hbm.at[idx_v.at[b].at[pl.ds(j * GDMA, GDMA)]],
                             rows_v.at[b].at[pl.ds(j * GDMA, GDMA)], sems[b])

    def drain_gathers(b):
        for j in range(N_GATH):
            pltpu.make_async_copy(
                table_hbm.at[idx_v.at[b].at[pl.ds(j * GDMA, GDMA)]],
                rows_v.at[b].at[pl.ds(j * GDMA, GDMA)], sems[b]).wait()

    def compact_and_store(g, b):
        for h in range(2):
            def row_body(q, carry):
                for r in range(4):
                    t = q * 4 + r
                    for k in range(EMB // LANES):
                        rows64_v[t, pl.ds(k * LANES, LANES)] = (
                            rows_v.at[b][h * HALF + t,
                                         pl.ds(k * LANES, LANES)])
                return carry

            lax.fori_loop(0, HALF // 4, row_body, 0)
            pltpu.sync_copy(
                rows64_v,
                out_hbm.at[pl.ds(tok_base + g * CHUNK + h * HALF, HALF)])

    stage_and_fire(0, 0)
    stage_and_fire(1, 1)

    def body(p, carry):
        for b in range(2):
            g = 2 * p + b
            drain_gathers(b)
            compact_and_store(g, b)
            stage_and_fire(g + 2, b)
        return carry

    lax.fori_loop(0, N_CHUNKS // 2 - 1, body, 0)

    for b in range(2):
        g = N_CHUNKS - 2 + b
        drain_gathers(b)
        compact_and_store(g, b)


def kernel(x, table):
    table_padded = jnp.pad(table, ((0, 0), (0, PAD_EMB - EMB)))
    out = _emb_lookup(x.astype(jnp.int32), table_padded)
    return out.reshape(BATCH, SEQ, EMB)


# speedup vs baseline: 1.3923x; 1.0038x over previous
import functools

import jax
import jax.numpy as jnp
from jax import lax
from jax.experimental import pallas as pl
from jax.experimental.pallas import tpu as pltpu
from jax.experimental.pallas import tpu_sc as plsc

VOCAB = 1000000
EMB = 64
PAD_EMB = 128
BATCH = 4096
SEQ = 200
NTOK = BATCH * SEQ

NC = 2
NS = 16
NW = NC * NS
ROWS_W = BATCH // NW
PER_W = NTOK // NW

GDMA = 128
CHUNK = 256
N_GATH = CHUNK // GDMA
N_CHUNKS = PER_W // CHUNK
LANES = 16
HALF = CHUNK // 2


@functools.partial(
    pl.kernel,
    mesh=plsc.VectorSubcoreMesh(core_axis_name="c", subcore_axis_name="s"),
    compiler_params=pltpu.CompilerParams(needs_layout_passes=False),
    out_type=jax.ShapeDtypeStruct((NTOK, EMB), jnp.float32),
    scratch_types=[
        pltpu.VMEM((ROWS_W // 2, SEQ), jnp.int32),
        pltpu.VMEM((2, CHUNK), jnp.int32),
        pltpu.VMEM((2, CHUNK, PAD_EMB), jnp.float32),
        pltpu.VMEM((2, HALF, EMB), jnp.float32),
        pltpu.SemaphoreType.DMA,
        pltpu.SemaphoreType.DMA,
        pltpu.SemaphoreType.DMA,
        pltpu.SemaphoreType.DMA,
    ],
)
def _emb_lookup(x_hbm, table_hbm, out_hbm, x_v, idx_v, rows_v, rows64_v,
                sem0, sem1, osem0, osem1):
    wid = lax.axis_index("s") * NC + lax.axis_index("c")
    tok_base = wid * PER_W
    sems = (sem0, sem1)
    osems = (osem0, osem1)
    x_half_rows = ROWS_W // 2
    switch_chunk = N_CHUNKS // 2

    pltpu.sync_copy(x_hbm.at[pl.ds(wid * ROWS_W, x_half_rows)], x_v)

    def stage_and_fire(g, b):
        g_t = jnp.int32(g)

        @pl.when(g_t == switch_chunk)
        def _():
            pltpu.sync_copy(
                x_hbm.at[pl.ds(wid * ROWS_W + x_half_rows, x_half_rows)], x_v)

        row_off = jnp.where(g_t >= switch_chunk, x_half_rows, 0)
        for k in range(CHUNK // LANES):
            u = g * CHUNK + k * LANES + lax.iota(jnp.int32, LANES)
            ids = plsc.load_gather(
                x_v, [lax.div(u, SEQ) - row_off, lax.rem(u, SEQ)])
            idx_v[b, pl.ds(k * LANES, LANES)] = ids
        for j in range(N_GATH):
            pltpu.async_copy(table_hbm.at[idx_v.at[b].at[pl.ds(j * GDMA, GDMA)]],
                             rows_v.at[b].at[pl.ds(j * GDMA, GDMA)], sems[b])

    def drain_gathers(b):
        for j in range(N_GATH):
            pltpu.make_async_copy(
                table_hbm.at[idx_v.at[b].at[pl.ds(j * GDMA, GDMA)]],
                rows_v.at[b].at[pl.ds(j * GDMA, GDMA)], sems[b]).wait()

    def wait_store(h):
        pltpu.make_async_copy(rows64_v.at[h],
                              out_hbm.at[pl.ds(tok_base, HALF)],
                              osems[h]).wait()

    def compact_and_store(g, b):
        g_t = jnp.int32(g)
        for h in range(2):
            @pl.when(g_t > 0)
            def _():
                wait_store(h)

            def row_body(q, carry):
                for r in range(4):
                    t = q * 4 + r
                    for k in range(EMB // LANES):
                        rows64_v[h, t, pl.ds(k * LANES, LANES)] = (
                            rows_v.at[b][h * HALF + t,
                                         pl.ds(k * LANES, LANES)])
                return carry

            lax.fori_loop(0, HALF // 4, row_body, 0)
            pltpu.async_copy(
                rows64_v.at[h],
                out_hbm.at[pl.ds(tok_base + g * CHUNK + h * HALF, HALF)],
                osems[h])

    stage_and_fire(0, 0)
    stage_and_fire(1, 1)

    def body(p, carry):
        for b in range(2):
            g = 2 * p + b
            drain_gathers(b)
            compact_and_store(g, b)
            stage_and_fire(g + 2, b)
        return carry

    lax.fori_loop(0, N_CHUNKS // 2 - 1, body, 0)

    for b in range(2):
        g = N_CHUNKS - 2 + b
        drain_gathers(b)
        compact_and_store(g, b)
    for h in range(2):
        wait_store(h)


def kernel(x, table):
    table_padded = jnp.pad(table, ((0, 0), (0, PAD_EMB - EMB)))
    out = _emb_lookup(x.astype(jnp.int32), table_padded)
    return out.reshape(BATCH, SEQ, EMB)
